# baseline (speedup 1.0000x reference)
import functools

import jax
import jax.numpy as jnp
from jax import lax
from jax.experimental import pallas as pl
from jax.experimental.pallas import tpu as pltpu
from jax.experimental.pallas import tpu_sc as plsc

D_MODEL = 64
VOCAB = 1000000
SCALE = 8.0

S_DIM = 4096
T_DIM = 200
NC, NS = 2, 16
NW = NC * NS
LANES = 16

VBLK = 128
N_VFULL = VOCAB // VBLK
V_TAIL = VOCAB - N_VFULL * VBLK
A_EXTRA = N_VFULL % NW
A_BASE = N_VFULL // NW

SBLK = S_DIM // NW


def _iota16():
    return lax.iota(jnp.int32, 16)


def _conv_kernel(wt_hbm, tail_hbm, w128_hbm, in_v, out_v, gsem, osem):
    wid = lax.axis_index("s") * NC + lax.axis_index("c")
    n_blk = A_BASE + jnp.where(wid < A_EXTRA, 1, 0)

    def blk_idx(k):
        return k * NW + wid

    def start_in(k, b):
        pltpu.async_copy(
            wt_hbm.at[:, pl.ds(blk_idx(k) * VBLK, VBLK)], in_v.at[b], gsem.at[b]
        )

    def wait_in(k, b):
        pltpu.make_async_copy(
            wt_hbm.at[:, pl.ds(blk_idx(k) * VBLK, VBLK)], in_v.at[b], gsem.at[b]
        ).wait()

    def start_out(k, b):
        pltpu.async_copy(
            out_v.at[b], w128_hbm.at[pl.ds(blk_idx(k) * (VBLK // 2), VBLK // 2)],
            osem.at[b],
        )

    def wait_out(k, b):
        pltpu.make_async_copy(
            out_v.at[b], w128_hbm.at[pl.ds(blk_idx(k) * (VBLK // 2), VBLK // 2)],
            osem.at[b],
        ).wait()

    lane = _iota16()
    row_idx = [lane + (c0 % D_MODEL) for c0 in range(0, 2 * D_MODEL, LANES)]

    def transpose_blk(rb, ob):
        @plsc.parallel_loop(0, VBLK // 2, 1, unroll=8)
        def _(q):
            for half in range(2):
                col = jnp.full((16,), 2 * q + half, jnp.int32)
                for c4 in range(4):
                    c0 = half * D_MODEL + c4 * LANES
                    v = plsc.load_gather(in_v.at[rb], [row_idx[c0 // LANES], col])
                    out_v[ob, q, pl.ds(c0, LANES)] = v

    start_in(0, 0)
    start_in(1, 1)

    def step(k, rb, ob):
        @pl.when(k + 2 < n_blk)
        def _():
            start_in(k + 2, (rb + 2) % 3)

        wait_in(k, rb)

        @pl.when(k >= 2)
        def _():
            wait_out(k - 2, ob)

        transpose_blk(rb, ob)
        start_out(k, ob)

    def group_body(g, carry):
        for i in range(6):
            step(6 * g + i, i % 3, i % 2)
        return carry

    lax.fori_loop(0, n_blk // 6, group_body, 0, unroll=False)
    tail0 = (n_blk // 6) * 6
    for i in range(5):
        @pl.when(tail0 + i < n_blk)
        def _(i=i):
            step(tail0 + i, i % 3, i % 2)

    @pl.when(n_blk % 2 == 0)
    def _():
        wait_out(n_blk - 2, 0)
        wait_out(n_blk - 1, 1)

    @pl.when(n_blk % 2 == 1)
    def _():
        wait_out(n_blk - 2, 1)
        wait_out(n_blk - 1, 0)

    @pl.when(wid == A_EXTRA)
    def _():
        pltpu.async_copy(tail_hbm, in_v.at[0], gsem.at[0]).wait()

        @plsc.parallel_loop(0, V_TAIL // 2, 1, unroll=8)
        def _(q):
            for half in range(2):
                col = jnp.full((16,), 2 * q + half, jnp.int32)
                for c4 in range(4):
                    c0 = half * D_MODEL + c4 * LANES
                    v = plsc.load_gather(in_v.at[0], [row_idx[c0 // LANES], col])
                    out_v[0, q, pl.ds(c0, LANES)] = v
        pltpu.async_copy(
            out_v.at[0, pl.ds(0, V_TAIL // 2)],
            w128_hbm.at[pl.ds(N_VFULL * (VBLK // 2), V_TAIL // 2)],
            osem.at[0],
        ).wait()


def _gather_kernel(xt_hbm, w128_hbm, out_hbm, idx_v, pidx_v, rows_v, ot_v,
                   gsem, osem):
    wid = lax.axis_index("s") * NC + lax.axis_index("c")
    s0 = wid * SBLK
    pltpu.sync_copy(xt_hbm.at[:, pl.ds(s0, SBLK)], idx_v)

    def pidx_body(t, carry):
        for l in range(SBLK // LANES):
            sl = pl.ds(l * LANES, LANES)
            pidx_v[t, sl] = jax.lax.shift_right_logical(idx_v[t, sl], 1)
        return carry

    lax.fori_loop(0, T_DIM, pidx_body, 0, unroll=False)

    def start_gather(t, b):
        pltpu.async_copy(w128_hbm.at[pidx_v.at[t]], rows_v.at[b], gsem.at[b])

    def wait_gather(t, b):
        pltpu.make_async_copy(
            w128_hbm.at[pidx_v.at[t]], rows_v.at[b], gsem.at[b]
        ).wait()

    def start_out(t, b):
        pltpu.async_copy(
            ot_v.at[b], out_hbm.at[t, :, pl.ds(s0, SBLK)], osem.at[b]
        )

    def wait_out(t, b):
        pltpu.make_async_copy(
            ot_v.at[b], out_hbm.at[t, :, pl.ds(s0, SBLK)], osem.at[b]
        ).wait()

    lane = _iota16()
    row_idx = [lane + l * LANES for l in range(SBLK // LANES)]

    def transpose_blk(t, rb, ob):
        hvecs = []
        for l in range(SBLK // LANES):
            sl = pl.ds(l * LANES, LANES)
            hvecs.append(
                jax.lax.shift_left(jnp.bitwise_and(idx_v[t, sl], 1), 6)
            )

        @plsc.parallel_loop(0, D_MODEL, 1, unroll=8)
        def _(j):
            for l in range(SBLK // LANES):
                col = hvecs[l] + j
                v = plsc.load_gather(rows_v.at[rb], [row_idx[l], col])
                ot_v[ob, j, pl.ds(l * LANES, LANES)] = v * SCALE

    start_gather(0, 0)
    start_gather(1, 1)

    def step(t, rb, ob):
        @pl.when(t + 2 < T_DIM)
        def _():
            start_gather(t + 2, (rb + 2) % 3)

        wait_gather(t, rb)

        @pl.when(t >= 2)
        def _():
            wait_out(t - 2, ob)

        transpose_blk(t, rb, ob)
        start_out(t, ob)

    def group_body(g, carry):
        for i in range(6):
            step(6 * g + i, i % 3, i % 2)
        return carry

    lax.fori_loop(0, T_DIM // 6, group_body, 0, unroll=False)
    for i in range(T_DIM % 6):
        step((T_DIM // 6) * 6 + i, i % 3, i % 2)
    wait_out(T_DIM - 2, 0)
    wait_out(T_DIM - 1, 1)


@jax.jit
def _emb(xt, wt):
    mesh = plsc.VectorSubcoreMesh(core_axis_name="c", subcore_axis_name="s")
    conv = functools.partial(
        pl.kernel,
        mesh=mesh,
        out_type=jax.ShapeDtypeStruct((VOCAB // 2, 2 * D_MODEL), jnp.float32),
        scratch_types=[
            pltpu.VMEM((3, D_MODEL, VBLK), jnp.float32),
            pltpu.VMEM((2, VBLK // 2, 2 * D_MODEL), jnp.float32),
            pltpu.SemaphoreType.DMA((3,)),
            pltpu.SemaphoreType.DMA((2,)),
        ],
        compiler_params=pltpu.CompilerParams(needs_layout_passes=False),
    )(_conv_kernel)
    tail_w = jnp.pad(wt[:, N_VFULL * VBLK:], ((0, 0), (0, VBLK - V_TAIL)))
    w128 = conv(wt, tail_w)

    gath = functools.partial(
        pl.kernel,
        mesh=mesh,
        out_type=jax.ShapeDtypeStruct((T_DIM, D_MODEL, S_DIM), jnp.float32),
        scratch_types=[
            pltpu.VMEM((T_DIM, SBLK), jnp.int32),
            pltpu.VMEM((T_DIM, SBLK), jnp.int32),
            pltpu.VMEM((3, SBLK, 2 * D_MODEL), jnp.float32),
            pltpu.VMEM((2, D_MODEL, SBLK), jnp.float32),
            pltpu.SemaphoreType.DMA((3,)),
            pltpu.SemaphoreType.DMA((2,)),
        ],
        compiler_params=pltpu.CompilerParams(needs_layout_passes=False),
    )(_gather_kernel)
    return gath(xt, w128)


def kernel(x, emb_weight):
    out3 = _emb(x.T, emb_weight.T)
    return out3.transpose(2, 0, 1)


# speedup vs baseline: 3.0587x; 3.0587x over previous
import functools

import jax
import jax.numpy as jnp
from jax import lax
from jax.experimental import pallas as pl
from jax.experimental.pallas import tpu as pltpu
from jax.experimental.pallas import tpu_sc as plsc

D_MODEL = 64
VOCAB = 1000000
SCALE = 8.0

S_DIM = 4096
T_DIM = 200
NC, NS = 2, 16
NW = NC * NS
LANES = 16

VBLK = 128
N_VFULL = VOCAB // VBLK
V_TAIL = VOCAB - N_VFULL * VBLK
A_EXTRA = N_VFULL % NW
A_BASE = N_VFULL // NW

SBLK = S_DIM // NW


def _iota16():
    return lax.iota(jnp.int32, 16)


PITCH = 129


def _conv_kernel(wt_hbm, tail_hbm, w128_hbm, in_v, in_p, out_v, gsem, osem):
    wid = lax.axis_index("s") * NC + lax.axis_index("c")
    n_blk = A_BASE + jnp.where(wid < A_EXTRA, 1, 0)

    def blk_idx(k):
        return k * NW + wid

    def start_in(k, b):
        pltpu.async_copy(
            wt_hbm.at[:, pl.ds(blk_idx(k) * VBLK, VBLK)], in_v.at[b], gsem.at[b]
        )

    def wait_in(k, b):
        pltpu.make_async_copy(
            wt_hbm.at[:, pl.ds(blk_idx(k) * VBLK, VBLK)], in_v.at[b], gsem.at[b]
        ).wait()

    def start_out(k, b):
        pltpu.async_copy(
            out_v.at[b], w128_hbm.at[pl.ds(blk_idx(k) * (VBLK // 2), VBLK // 2)],
            osem.at[b],
        )

    def wait_out(k, b):
        pltpu.make_async_copy(
            out_v.at[b], w128_hbm.at[pl.ds(blk_idx(k) * (VBLK // 2), VBLK // 2)],
            osem.at[b],
        ).wait()

    lane = _iota16()
    rp_idx = [(lane + c0) * PITCH for c0 in range(0, D_MODEL, LANES)]

    def transpose_blk(rb, ob):
        @plsc.parallel_loop(0, D_MODEL, 1, unroll=4)
        def _(r):
            base = r * PITCH
            for l in range(VBLK // LANES):
                in_p[pl.ds(base + l * LANES, LANES)] = in_v[
                    rb, r, pl.ds(l * LANES, LANES)
                ]

        @plsc.parallel_loop(0, VBLK // 2, 1, unroll=8)
        def _(q):
            for half in range(2):
                for c4 in range(4):
                    c0 = half * D_MODEL + c4 * LANES
                    v = plsc.load_gather(
                        in_p, [rp_idx[c4] + (2 * q + half)]
                    )
                    out_v[ob, q, pl.ds(c0, LANES)] = v

    start_in(0, 0)
    start_in(1, 1)

    def step(k, rb, ob):
        @pl.when(k + 2 < n_blk)
        def _():
            start_in(k + 2, (rb + 2) % 3)

        wait_in(k, rb)

        @pl.when(k >= 2)
        def _():
            wait_out(k - 2, ob)

        transpose_blk(rb, ob)
        start_out(k, ob)

    def group_body(g, carry):
        for i in range(6):
            step(6 * g + i, i % 3, i % 2)
        return carry

    lax.fori_loop(0, n_blk // 6, group_body, 0, unroll=False)
    tail0 = (n_blk // 6) * 6
    for i in range(5):
        @pl.when(tail0 + i < n_blk)
        def _(i=i):
            step(tail0 + i, i % 3, i % 2)

    @pl.when(n_blk % 2 == 0)
    def _():
        wait_out(n_blk - 2, 0)
        wait_out(n_blk - 1, 1)

    @pl.when(n_blk % 2 == 1)
    def _():
        wait_out(n_blk - 2, 1)
        wait_out(n_blk - 1, 0)

    @pl.when(wid == A_EXTRA)
    def _():
        pltpu.async_copy(tail_hbm, in_v.at[0], gsem.at[0]).wait()
        transpose_blk(0, 0)
        pltpu.async_copy(
            out_v.at[0, pl.ds(0, V_TAIL // 2)],
            w128_hbm.at[pl.ds(N_VFULL * (VBLK // 2), V_TAIL // 2)],
            osem.at[0],
        ).wait()


def _gather_kernel(xt_hbm, w128_hbm, out_hbm, idx_v, pidx_v, rows_v, rows_p,
                   ot_v, gsem, osem):
    wid = lax.axis_index("s") * NC + lax.axis_index("c")
    s0 = wid * SBLK
    pltpu.sync_copy(xt_hbm.at[:, pl.ds(s0, SBLK)], idx_v)

    def build_pidx(t, b):
        for l in range(SBLK // LANES):
            sl = pl.ds(l * LANES, LANES)
            pidx_v[b, sl] = jax.lax.shift_right_logical(idx_v[t, sl], 1)

    def start_gather(t, b):
        build_pidx(t, b)
        pltpu.async_copy(w128_hbm.at[pidx_v.at[b]], rows_v.at[b], gsem.at[b])

    def wait_gather(t, b):
        pltpu.make_async_copy(
            w128_hbm.at[pidx_v.at[b]], rows_v.at[b], gsem.at[b]
        ).wait()

    def start_out(t, b):
        pltpu.async_copy(
            ot_v.at[b], out_hbm.at[t, :, pl.ds(s0, SBLK)], osem.at[b]
        )

    def wait_out(t, b):
        pltpu.make_async_copy(
            ot_v.at[b], out_hbm.at[t, :, pl.ds(s0, SBLK)], osem.at[b]
        ).wait()

    lane = _iota16()
    rp_idx = [(lane + l * LANES) * PITCH for l in range(SBLK // LANES)]

    def transpose_blk(t, rb, ob):
        @plsc.parallel_loop(0, SBLK, 1, unroll=4)
        def _(s):
            base = s * PITCH
            for l in range(2 * D_MODEL // LANES):
                rows_p[pl.ds(base + l * LANES, LANES)] = rows_v[
                    rb, s, pl.ds(l * LANES, LANES)
                ]

        base_idx = []
        for l in range(SBLK // LANES):
            sl = pl.ds(l * LANES, LANES)
            h64 = jax.lax.shift_left(jnp.bitwise_and(idx_v[t, sl], 1), 6)
            base_idx.append(rp_idx[l] + h64)

        @plsc.parallel_loop(0, D_MODEL, 1, unroll=8)
        def _(j):
            for l in range(SBLK // LANES):
                v = plsc.load_gather(rows_p, [base_idx[l] + j])
                ot_v[ob, j, pl.ds(l * LANES, LANES)] = v * SCALE

    start_gather(0, 0)
    start_gather(1, 1)

    def step(t, rb, ob):
        @pl.when(t + 2 < T_DIM)
        def _():
            start_gather(t + 2, (rb + 2) % 3)

        wait_gather(t, rb)

        @pl.when(t >= 2)
        def _():
            wait_out(t - 2, ob)

        transpose_blk(t, rb, ob)
        start_out(t, ob)

    def group_body(g, carry):
        for i in range(6):
            step(6 * g + i, i % 3, i % 2)
        return carry

    lax.fori_loop(0, T_DIM // 6, group_body, 0, unroll=False)
    for i in range(T_DIM % 6):
        step((T_DIM // 6) * 6 + i, i % 3, i % 2)
    wait_out(T_DIM - 2, 0)
    wait_out(T_DIM - 1, 1)


@jax.jit
def _emb(xt, wt):
    mesh = plsc.VectorSubcoreMesh(core_axis_name="c", subcore_axis_name="s")
    conv = functools.partial(
        pl.kernel,
        mesh=mesh,
        out_type=jax.ShapeDtypeStruct((VOCAB // 2, 2 * D_MODEL), jnp.float32),
        scratch_types=[
            pltpu.VMEM((3, D_MODEL, VBLK), jnp.float32),
            pltpu.VMEM((D_MODEL * PITCH,), jnp.float32),
            pltpu.VMEM((2, VBLK // 2, 2 * D_MODEL), jnp.float32),
            pltpu.SemaphoreType.DMA((3,)),
            pltpu.SemaphoreType.DMA((2,)),
        ],
        compiler_params=pltpu.CompilerParams(needs_layout_passes=False),
    )(_conv_kernel)
    tail_w = jnp.pad(wt[:, N_VFULL * VBLK:], ((0, 0), (0, VBLK - V_TAIL)))
    w128 = conv(wt, tail_w)

    gath = functools.partial(
        pl.kernel,
        mesh=mesh,
        out_type=jax.ShapeDtypeStruct((T_DIM, D_MODEL, S_DIM), jnp.float32),
        scratch_types=[
            pltpu.VMEM((T_DIM, SBLK), jnp.int32),
            pltpu.VMEM((3, SBLK), jnp.int32),
            pltpu.VMEM((3, SBLK, 2 * D_MODEL), jnp.float32),
            pltpu.VMEM((SBLK * PITCH,), jnp.float32),
            pltpu.VMEM((2, D_MODEL, SBLK), jnp.float32),
            pltpu.SemaphoreType.DMA((3,)),
            pltpu.SemaphoreType.DMA((2,)),
        ],
        compiler_params=pltpu.CompilerParams(needs_layout_passes=False),
    )(_gather_kernel)
    return gath(xt, w128)


def kernel(x, emb_weight):
    out3 = _emb(x.T, emb_weight.T)
    return out3.transpose(2, 0, 1)


# speedup vs baseline: 3.3771x; 1.1041x over previous
import functools

import jax
import jax.numpy as jnp
from jax import lax
from jax.experimental import pallas as pl
from jax.experimental.pallas import tpu as pltpu
from jax.experimental.pallas import tpu_sc as plsc

D_MODEL = 64
VOCAB = 1000000
SCALE = 8.0

S_DIM = 4096
T_DIM = 200
NC, NS = 2, 16
NW = NC * NS
LANES = 16

VBLK = 128
N_VFULL = VOCAB // VBLK
V_TAIL = VOCAB - N_VFULL * VBLK
A_EXTRA = N_VFULL % NW
A_BASE = N_VFULL // NW

SBLK = S_DIM // NW


def _iota16():
    return lax.iota(jnp.int32, 16)


PITCH = 129


def _conv_kernel(wt_hbm, tail_hbm, wlin_hbm, in_v, in_p, out_v, gsem, osem):
    wid = lax.axis_index("s") * NC + lax.axis_index("c")
    n_blk = A_BASE + jnp.where(wid < A_EXTRA, 1, 0)

    def blk_idx(k):
        return k * NW + wid

    def start_in(k, b):
        pltpu.async_copy(
            wt_hbm.at[:, pl.ds(blk_idx(k) * VBLK, VBLK)], in_v.at[b], gsem.at[b]
        )

    def wait_in(k, b):
        pltpu.make_async_copy(
            wt_hbm.at[:, pl.ds(blk_idx(k) * VBLK, VBLK)], in_v.at[b], gsem.at[b]
        ).wait()

    BW = VBLK * D_MODEL

    def start_out(k, b):
        pltpu.async_copy(
            out_v.at[b], wlin_hbm.at[pl.ds(blk_idx(k) * BW, BW)], osem.at[b]
        )

    def wait_out(k, b):
        pltpu.make_async_copy(
            out_v.at[b], wlin_hbm.at[pl.ds(blk_idx(k) * BW, BW)], osem.at[b]
        ).wait()

    lane = _iota16()
    rp_idx = [(lane + c0) * PITCH for c0 in range(0, D_MODEL, LANES)]

    def transpose_blk(rb, ob):
        @plsc.parallel_loop(0, D_MODEL, 1, unroll=4)
        def _(r):
            base = r * PITCH
            for l in range(VBLK // LANES):
                in_p[pl.ds(base + l * LANES, LANES)] = in_v[
                    rb, r, pl.ds(l * LANES, LANES)
                ]

        @plsc.parallel_loop(0, VBLK, 1, unroll=8)
        def _(v_):
            for c4 in range(4):
                x = plsc.load_gather(in_p, [rp_idx[c4] + v_])
                out_v[ob, pl.ds(v_ * D_MODEL + c4 * LANES, LANES)] = x

    start_in(0, 0)
    start_in(1, 1)

    def step(k, rb, ob):
        @pl.when(k + 2 < n_blk)
        def _():
            start_in(k + 2, (rb + 2) % 3)

        wait_in(k, rb)

        @pl.when(k >= 2)
        def _():
            wait_out(k - 2, ob)

        transpose_blk(rb, ob)
        start_out(k, ob)

    def group_body(g, carry):
        for i in range(6):
            step(6 * g + i, i % 3, i % 2)
        return carry

    lax.fori_loop(0, n_blk // 6, group_body, 0, unroll=False)
    tail0 = (n_blk // 6) * 6
    for i in range(5):
        @pl.when(tail0 + i < n_blk)
        def _(i=i):
            step(tail0 + i, i % 3, i % 2)

    @pl.when(n_blk % 2 == 0)
    def _():
        wait_out(n_blk - 2, 0)
        wait_out(n_blk - 1, 1)

    @pl.when(n_blk % 2 == 1)
    def _():
        wait_out(n_blk - 2, 1)
        wait_out(n_blk - 1, 0)

    @pl.when(wid == A_EXTRA)
    def _():
        pltpu.async_copy(tail_hbm, in_v.at[0], gsem.at[0]).wait()
        transpose_blk(0, 0)
        pltpu.async_copy(
            out_v.at[0, pl.ds(0, V_TAIL * D_MODEL)],
            wlin_hbm.at[pl.ds(N_VFULL * BW, V_TAIL * D_MODEL)],
            osem.at[0],
        ).wait()


GPITCH = 65


def _gather_kernel(xt_hbm, w_hbm, out_hbm, idx_v, rows_v, rows_p,
                   ot_v, gsem, osem):
    wid = lax.axis_index("s") * NC + lax.axis_index("c")
    s0 = wid * SBLK
    pltpu.sync_copy(xt_hbm.at[:, pl.ds(s0, SBLK)], idx_v)

    def start_gather(t, b):
        pltpu.async_copy(w_hbm.at[idx_v.at[t]], rows_v.at[b], gsem.at[b])

    def wait_gather(t, b):
        pltpu.make_async_copy(
            w_hbm.at[idx_v.at[t]], rows_v.at[b], gsem.at[b]
        ).wait()

    def start_out(t, b):
        pltpu.async_copy(
            ot_v.at[b], out_hbm.at[t, :, wid, :, :], osem.at[b]
        )

    def wait_out(t, b):
        pltpu.make_async_copy(
            ot_v.at[b], out_hbm.at[t, :, wid, :, :], osem.at[b]
        ).wait()

    lane = _iota16()
    rp_idx = [(lane + l * LANES) * GPITCH for l in range(SBLK // LANES)]

    def transpose_blk(t, rb, ob):
        @plsc.parallel_loop(0, SBLK, 1, unroll=4)
        def _(s):
            base = s * GPITCH
            for l in range(D_MODEL // LANES):
                rows_p[pl.ds(base + l * LANES, LANES)] = rows_v[
                    rb, s, pl.ds(l * LANES, LANES)
                ]

        @plsc.parallel_loop(0, D_MODEL, 1, unroll=8)
        def _(j):
            for l in range(SBLK // LANES):
                v = plsc.load_gather(rows_p, [rp_idx[l] + j])
                ot_v[ob, j // 8, j % 8, pl.ds(l * LANES, LANES)] = v * SCALE

    start_gather(0, 0)
    start_gather(1, 1)

    def step(t, rb, ob):
        @pl.when(t + 2 < T_DIM)
        def _():
            start_gather(t + 2, (rb + 2) % 3)

        wait_gather(t, rb)

        @pl.when(t >= 2)
        def _():
            wait_out(t - 2, ob)

        transpose_blk(t, rb, ob)
        start_out(t, ob)

    def group_body(g, carry):
        for i in range(6):
            step(6 * g + i, i % 3, i % 2)
        return carry

    lax.fori_loop(0, T_DIM // 6, group_body, 0, unroll=False)
    for i in range(T_DIM % 6):
        step((T_DIM // 6) * 6 + i, i % 3, i % 2)
    wait_out(T_DIM - 2, 0)
    wait_out(T_DIM - 1, 1)


@jax.jit
def _emb(xt, wt):
    mesh = plsc.VectorSubcoreMesh(core_axis_name="c", subcore_axis_name="s")
    conv = functools.partial(
        pl.kernel,
        mesh=mesh,
        out_type=jax.ShapeDtypeStruct((VOCAB * D_MODEL,), jnp.float32),
        scratch_types=[
            pltpu.VMEM((3, D_MODEL, VBLK), jnp.float32),
            pltpu.VMEM((D_MODEL * PITCH,), jnp.float32),
            pltpu.VMEM((2, VBLK * D_MODEL), jnp.float32),
            pltpu.SemaphoreType.DMA((3,)),
            pltpu.SemaphoreType.DMA((2,)),
        ],
        compiler_params=pltpu.CompilerParams(needs_layout_passes=False),
    )(_conv_kernel)
    tail_w = jnp.pad(wt[:, N_VFULL * VBLK:], ((0, 0), (0, VBLK - V_TAIL)))
    w_lin = conv(wt, tail_w)
    w2 = w_lin.reshape(VOCAB, D_MODEL)

    gath = functools.partial(
        pl.kernel,
        mesh=mesh,
        out_type=jax.ShapeDtypeStruct(
            (T_DIM, D_MODEL // 8, S_DIM // SBLK, 8, SBLK), jnp.float32
        ),
        scratch_types=[
            pltpu.VMEM((T_DIM, SBLK), jnp.int32),
            pltpu.VMEM((3, SBLK, D_MODEL), jnp.float32),
            pltpu.VMEM((SBLK * GPITCH,), jnp.float32),
            pltpu.VMEM((2, D_MODEL // 8, 8, SBLK), jnp.float32),
            pltpu.SemaphoreType.DMA((3,)),
            pltpu.SemaphoreType.DMA((2,)),
        ],
        compiler_params=pltpu.CompilerParams(
            use_tc_tiling_on_sc=False, needs_layout_passes=False
        ),
    )(_gather_kernel)
    out5 = gath(xt, w2)
    return out5.transpose(2, 4, 0, 1, 3).reshape(S_DIM, T_DIM, D_MODEL)


def kernel(x, emb_weight):
    return _emb(x.T, emb_weight.T)
